# speedup vs baseline: 1.0637x; 1.0637x over previous
import functools
import jax
import jax.numpy as jnp
from jax.experimental import pallas as pl

_N = 10000
_E = 320000
_IN = 128
_H = 8
_D = 16
_HD = _H * _D

_BN = 400


def _mm_body(x_ref, a_ref, hs_ref, at_ref):
    xb = x_ref[...]
    out = jnp.dot(xb, a_ref[...], preferred_element_type=jnp.float32)
    hs_ref[...] = out[:, : 2 * _HD]
    at_ref[...] = out[:, 2 * _HD :]


def _dense_proj(x, A):
    n = x.shape[0]
    grid = (n // _BN,)
    return pl.pallas_call(
        _mm_body,
        grid=grid,
        in_specs=[
            pl.BlockSpec((_BN, _IN), lambda i: (i, 0)),
            pl.BlockSpec((_IN, 2 * _HD + 2 * _H), lambda i: (0, 0)),
        ],
        out_specs=[
            pl.BlockSpec((_BN, 2 * _HD), lambda i: (i, 0)),
            pl.BlockSpec((_BN, 2 * _H), lambda i: (i, 0)),
        ],
        out_shape=[
            jax.ShapeDtypeStruct((n, 2 * _HD), jnp.float32),
            jax.ShapeDtypeStruct((n, 2 * _H), jnp.float32),
        ],
    )(x, A)


def kernel(x, edge_index, W, att_src, att_dst, bias, skip_W, gamma, beta):
    n = x.shape[0]
    src = edge_index[0]
    dst = edge_index[1]

    A_src = jnp.einsum("hdk,hd->kh", W.reshape(_H, _D, _IN), att_src)
    A_dst = jnp.einsum("hdk,hd->kh", W.reshape(_H, _D, _IN), att_dst)
    A = jnp.concatenate([W.T, skip_W.T, A_src, A_dst], axis=1)

    hs, at = _dense_proj(x, A)
    h = hs[:, :_HD]
    skip = hs[:, _HD:]
    a_src = at[:, :_H]
    a_dst = at[:, _H:]

    M = jnp.max(a_src, axis=0) + jnp.max(a_dst, axis=0)
    M = jnp.maximum(M, 0.2 * M)

    e = a_src[src] + a_dst[dst]
    e = jnp.where(e > 0, e, 0.2 * e)
    w = jnp.exp(e - M[None, :])
    denom = jax.ops.segment_sum(w, dst, num_segments=n)
    alpha = w / (denom[dst] + 1e-16)
    msg = h[src].reshape(_E, _H, _D) * alpha[:, :, None]
    gat = jax.ops.segment_sum(msg, dst, num_segments=n).reshape(n, _HD)

    pre = gat + bias + 0.1 * skip
    mu = jnp.mean(pre, axis=-1, keepdims=True)
    var = jnp.mean((pre - mu) ** 2, axis=-1, keepdims=True)
    y = (pre - mu) / jnp.sqrt(var + 1e-5) * gamma + beta
    return jnp.where(y > 0, y, jnp.expm1(y))


# speedup vs baseline: 40.3829x; 37.9654x over previous
import functools
import jax
import jax.numpy as jnp
from jax import lax
from jax.experimental import pallas as pl
from jax.experimental.pallas import tpu as pltpu
from jax.experimental.pallas import tpu_sc as plsc

_N = 10000
_E = 320000
_IN = 128
_H = 8
_D = 16
_HD = _H * _D
_HQ = _HD // 4
_HPQ = _H // 4

_BN = 400
_NC = 2
_NS = 16
_NW = _NC * _NS
_EW = _E // _NW
_B = 200
_RT = 624
_RZ = 104
_RREM = _N - _NS * _RT


def _proj_body(x_ref, aw0_ref, aw1_ref, aw2_ref, aw3_ref, as_ref,
               a1_ref, a2_ref,
               h0_ref, h1_ref, h2_ref, h3_ref, sk_ref, at1_ref, at2_ref):
    xb = x_ref[...]
    for aw_ref, h_ref in ((aw0_ref, h0_ref), (aw1_ref, h1_ref),
                          (aw2_ref, h2_ref), (aw3_ref, h3_ref)):
        h_ref[...] = jnp.dot(xb, aw_ref[...],
                             preferred_element_type=jnp.float32)
    sk_ref[...] = jnp.dot(xb, as_ref[...], preferred_element_type=jnp.float32)
    at1_ref[...] = jnp.dot(xb, a1_ref[...], preferred_element_type=jnp.float32)
    at2_ref[...] = jnp.dot(xb, a2_ref[...], preferred_element_type=jnp.float32)


def _dense_proj(x, AWq, AS, A1, A2):
    n = x.shape[0]
    return pl.pallas_call(
        _proj_body,
        grid=(n // _BN,),
        in_specs=[
            pl.BlockSpec((_BN, _IN), lambda i: (i, 0)),
        ] + [pl.BlockSpec((_IN, _HQ), lambda i: (0, 0))] * 4 + [
            pl.BlockSpec((_IN, _HD), lambda i: (0, 0)),
            pl.BlockSpec((_IN, 16), lambda i: (0, 0)),
            pl.BlockSpec((_IN, 16), lambda i: (0, 0)),
        ],
        out_specs=[
            pl.BlockSpec((_BN, _HQ), lambda i: (i, 0)) for _ in range(4)
        ] + [
            pl.BlockSpec((_BN, _HD), lambda i: (i, 0)),
            pl.BlockSpec((_BN, 16), lambda i: (i, 0)),
            pl.BlockSpec((_BN, 16), lambda i: (i, 0)),
        ],
        out_shape=[
            jax.ShapeDtypeStruct((n, _HQ), jnp.float32) for _ in range(4)
        ] + [
            jax.ShapeDtypeStruct((n, _HD), jnp.float32),
            jax.ShapeDtypeStruct((n, 16), jnp.float32),
            jax.ShapeDtypeStruct((n, 16), jnp.float32),
        ],
    )(x, *AWq, AS, A1, A2)


def _zero_vmem(buf, width):
    @pl.loop(0, _B)
    def _(i):
        for j in range(width // _D):
            buf[i, pl.ds(j * _D, _D)] = jnp.zeros((_D,), jnp.float32)


def _zero_rows(buf, acc, row0, s):
    for q in range(6):
        pltpu.sync_copy(buf.at[pl.ds(0, _RZ)],
                        acc.at[pl.ds(row0 + q * _RZ, _RZ)])

    @pl.when(s == _NS - 1)
    def _():
        pltpu.sync_copy(buf.at[pl.ds(0, _RREM)],
                        acc.at[pl.ds(_NS * _RT, _RREM)])


def _dump_rows(acc, hbm, c, row0, s):
    pltpu.sync_copy(acc.at[pl.ds(row0, _RT)], hbm.at[c, pl.ds(row0, _RT)])

    @pl.when(s == _NS - 1)
    def _():
        pltpu.sync_copy(acc.at[pl.ds(_NS * _RT, _RREM)],
                        hbm.at[c, pl.ds(_NS * _RT, _RREM)])


def _sc_body(src_hbm, dst_hbm, at1_hbm, at2_hbm,
             h0_hbm, h1_hbm, h2_hbm, h3_hbm, m_hbm,
             outp_hbm, denp_hbm,
             sidx, didx, b1, b2, dbuf, wres, hbuf, m_v, out_sp, den_sp):
    c = lax.axis_index("c")
    s = lax.axis_index("s")
    wid = c * _NS + s
    ebase = wid * _EW
    row0 = s * _RT

    _zero_vmem(hbuf, _HQ)
    _zero_vmem(dbuf, _D)
    _zero_rows(hbuf, out_sp, row0, s)
    _zero_rows(dbuf, den_sp, row0, s)
    pltpu.sync_copy(m_hbm, m_v)
    plsc.subcore_barrier()

    mv = m_v[...]
    lane = lax.iota(jnp.int32, 16)
    lo_mask = lane < 8

    @pl.loop(0, _EW, step=_B)
    def _(k):
        eb = ebase + k
        pltpu.sync_copy(src_hbm.at[pl.ds(eb, _B)], sidx)
        pltpu.sync_copy(dst_hbm.at[pl.ds(eb, _B)], didx)
        pltpu.sync_copy(at1_hbm.at[sidx], b1)
        pltpu.sync_copy(at2_hbm.at[didx], b2)

        @pl.loop(0, _B, step=2)
        def _(j):
            e1 = b1[j, :] + b2[j, :]
            e1 = jnp.maximum(e1, 0.2 * e1)
            w1 = jnp.exp(e1 - mv)
            e2 = b1[j + 1, :] + b2[j + 1, :]
            e2 = jnp.maximum(e2, 0.2 * e2)
            w2 = jnp.exp(e2 - mv)
            dbuf[j, :] = w1
            dbuf[j + 1, :] = w2
            wres[pl.ds((k + j) * _H, 16)] = jnp.where(lo_mask, w1, w2)

        pltpu.sync_copy(dbuf, den_sp.at[didx], add=True)

    for qt, h_hbm in enumerate((h0_hbm, h1_hbm, h2_hbm, h3_hbm)):
        @pl.loop(0, _EW, step=_B)
        def _(k):
            eb = ebase + k
            pltpu.sync_copy(src_hbm.at[pl.ds(eb, _B)], sidx)
            pltpu.sync_copy(dst_hbm.at[pl.ds(eb, _B)], didx)
            pltpu.sync_copy(h_hbm.at[sidx], hbuf)

            @pl.loop(0, _B, step=2)
            def _(e):
                wrow = wres[pl.ds((k + e) * _H, 16)]
                for ee in range(2):
                    for hh in range(_HPQ):
                        wv = wrow[ee * _H + qt * _HPQ + hh]
                        hbuf[e + ee, pl.ds(hh * _D, _D)] = (
                            hbuf[e + ee, pl.ds(hh * _D, _D)] * wv)

            pltpu.sync_copy(hbuf, out_sp.at[didx], add=True)

        plsc.subcore_barrier()
        _dump_rows(out_sp, outp_hbm.at[qt], c, row0, s)

        if qt < 3:
            _zero_vmem(hbuf, _HQ)
            _zero_rows(hbuf, out_sp, row0, s)
            plsc.subcore_barrier()

    _dump_rows(den_sp, denp_hbm, c, row0, s)


def _sc_edge(src, dst, at1, at2, hq, m16):
    mesh = plsc.VectorSubcoreMesh(core_axis_name="c", subcore_axis_name="s")

    k = pl.kernel(
        _sc_body,
        mesh=mesh,
        compiler_params=pltpu.CompilerParams(use_tc_tiling_on_sc=False),
        out_type=[
            jax.ShapeDtypeStruct((4, _NC, _N, _HQ), jnp.float32),
            jax.ShapeDtypeStruct((_NC, _N, _D), jnp.float32),
        ],
        scratch_types=[
            pltpu.VMEM((_B,), jnp.int32),
            pltpu.VMEM((_B,), jnp.int32),
            pltpu.VMEM((_B, _D), jnp.float32),
            pltpu.VMEM((_B, _D), jnp.float32),
            pltpu.VMEM((_B, _D), jnp.float32),
            pltpu.VMEM((_EW * _H,), jnp.float32),
            pltpu.VMEM((_B, _HQ), jnp.float32),
            pltpu.VMEM((16,), jnp.float32),
            pltpu.VMEM_SHARED((_N, _HQ), jnp.float32),
            pltpu.VMEM_SHARED((_N, _D), jnp.float32),
        ],
    )
    return k(src, dst, at1, at2, *hq, m16)


def _fin_body(o00, o01, o10, o11, o20, o21, o30, o31, d0_ref, d1_ref,
              skip_ref, bias_ref, gamma_ref, beta_ref, y_ref):
    den = d0_ref[...] + d1_ref[...]
    r_row = jax.lax.broadcasted_iota(jnp.int32, (16, _HD), 0)
    r_col = jax.lax.broadcasted_iota(jnp.int32, (16, _HD), 1)
    R = (r_col // _D == r_row).astype(jnp.float32)
    drep = jnp.dot(den, R, preferred_element_type=jnp.float32)
    gat = jnp.concatenate(
        [o00[...] + o01[...], o10[...] + o11[...],
         o20[...] + o21[...], o30[...] + o31[...]], axis=1)
    pre = gat / (drep + 1e-16) + bias_ref[...] + 0.1 * skip_ref[...]
    mu = jnp.mean(pre, axis=-1, keepdims=True)
    var = jnp.mean((pre - mu) ** 2, axis=-1, keepdims=True)
    y = (pre - mu) * jax.lax.rsqrt(var + 1e-5) * gamma_ref[...] + beta_ref[...]
    y_ref[...] = jnp.where(y > 0, y, jnp.exp(jnp.minimum(y, 0.0)) - 1.0)


def _finalize(oparts, d0, d1, skip, bias, gamma, beta):
    n = skip.shape[0]
    vec = lambda i: (0, 0)
    quarter = pl.BlockSpec((_BN, _HQ), lambda i: (i, 0))
    return pl.pallas_call(
        _fin_body,
        grid=(n // _BN,),
        in_specs=[quarter] * 8 + [
            pl.BlockSpec((_BN, _D), lambda i: (i, 0)),
            pl.BlockSpec((_BN, _D), lambda i: (i, 0)),
            pl.BlockSpec((_BN, _HD), lambda i: (i, 0)),
            pl.BlockSpec((1, _HD), vec),
            pl.BlockSpec((1, _HD), vec),
            pl.BlockSpec((1, _HD), vec),
        ],
        out_specs=pl.BlockSpec((_BN, _HD), lambda i: (i, 0)),
        out_shape=jax.ShapeDtypeStruct((n, _HD), jnp.float32),
    )(*oparts, d0, d1, skip, bias, gamma, beta)


def kernel(x, edge_index, W, att_src, att_dst, bias, skip_W, gamma, beta):
    src = edge_index[0]
    dst = edge_index[1]

    Wr = W.reshape(_H, _D, _IN)
    A_src = jnp.einsum("hdk,hd->kh", Wr, att_src)
    A_dst = jnp.einsum("hdk,hd->kh", Wr, att_dst)
    A1 = jnp.concatenate([A_src, A_src], axis=1)
    A2 = jnp.concatenate([A_dst, A_dst], axis=1)
    WT = W.T
    AWq = [WT[:, q * _HQ:(q + 1) * _HQ] for q in range(4)]

    h0, h1, h2, h3, skip, at1, at2 = _dense_proj(x, AWq, skip_W.T, A1, A2)

    M = jnp.max(at1, axis=0) + jnp.max(at2, axis=0)
    M16 = jnp.maximum(M, 0.2 * M)

    outp, denp = _sc_edge(src, dst, at1, at2, (h0, h1, h2, h3), M16)

    oparts = [outp[qt, cc] for qt in range(4) for cc in range(2)]
    y = _finalize(oparts, denp[0], denp[1], skip,
                  bias.reshape(1, _HD), gamma.reshape(1, _HD),
                  beta.reshape(1, _HD))
    return y


# speedup vs baseline: 44.6512x; 1.1057x over previous
import functools
import jax
import jax.numpy as jnp
from jax import lax
from jax.experimental import pallas as pl
from jax.experimental.pallas import tpu as pltpu
from jax.experimental.pallas import tpu_sc as plsc

_N = 10000
_E = 320000
_IN = 128
_H = 8
_D = 16
_HD = _H * _D
_HQ = _HD // 4
_HPQ = _H // 4

_BN = 400
_NC = 2
_NS = 16
_NW = _NC * _NS
_EW = _E // _NW
_B = 200
_RT = 624
_RZ = 104
_RREM = _N - _NS * _RT


def _proj_body(x_ref, aw0_ref, aw1_ref, aw2_ref, aw3_ref, as_ref,
               a1_ref, a2_ref,
               h0_ref, h1_ref, h2_ref, h3_ref, sk_ref, at1_ref, at2_ref):
    xb = x_ref[...]
    for aw_ref, h_ref in ((aw0_ref, h0_ref), (aw1_ref, h1_ref),
                          (aw2_ref, h2_ref), (aw3_ref, h3_ref)):
        h_ref[...] = jnp.dot(xb, aw_ref[...],
                             preferred_element_type=jnp.float32)
    sk_ref[...] = jnp.dot(xb, as_ref[...], preferred_element_type=jnp.float32)
    at1_ref[...] = jnp.dot(xb, a1_ref[...], preferred_element_type=jnp.float32)
    at2_ref[...] = jnp.dot(xb, a2_ref[...], preferred_element_type=jnp.float32)


def _dense_proj(x, AWq, AS, A1, A2):
    n = x.shape[0]
    return pl.pallas_call(
        _proj_body,
        grid=(n // _BN,),
        in_specs=[
            pl.BlockSpec((_BN, _IN), lambda i: (i, 0)),
        ] + [pl.BlockSpec((_IN, _HQ), lambda i: (0, 0))] * 4 + [
            pl.BlockSpec((_IN, _HD), lambda i: (0, 0)),
            pl.BlockSpec((_IN, 16), lambda i: (0, 0)),
            pl.BlockSpec((_IN, 16), lambda i: (0, 0)),
        ],
        out_specs=[
            pl.BlockSpec((_BN, _HQ), lambda i: (i, 0)) for _ in range(4)
        ] + [
            pl.BlockSpec((_BN, _HD), lambda i: (i, 0)),
            pl.BlockSpec((_BN, 16), lambda i: (i, 0)),
            pl.BlockSpec((_BN, 16), lambda i: (i, 0)),
        ],
        out_shape=[
            jax.ShapeDtypeStruct((n, _HQ), jnp.float32) for _ in range(4)
        ] + [
            jax.ShapeDtypeStruct((n, _HD), jnp.float32),
            jax.ShapeDtypeStruct((n, 16), jnp.float32),
            jax.ShapeDtypeStruct((n, 16), jnp.float32),
        ],
    )(x, *AWq, AS, A1, A2)


def _zero_vmem(buf, width):
    @pl.loop(0, _B)
    def _(i):
        for j in range(width // _D):
            buf[i, pl.ds(j * _D, _D)] = jnp.zeros((_D,), jnp.float32)


def _zero_rows(buf, acc, row0, s):
    for q in range(6):
        pltpu.sync_copy(buf.at[pl.ds(0, _RZ)],
                        acc.at[pl.ds(row0 + q * _RZ, _RZ)])

    @pl.when(s == _NS - 1)
    def _():
        pltpu.sync_copy(buf.at[pl.ds(0, _RREM)],
                        acc.at[pl.ds(_NS * _RT, _RREM)])


def _dump_rows(acc, hbm, c, row0, s):
    pltpu.sync_copy(acc.at[pl.ds(row0, _RT)], hbm.at[c, pl.ds(row0, _RT)])

    @pl.when(s == _NS - 1)
    def _():
        pltpu.sync_copy(acc.at[pl.ds(_NS * _RT, _RREM)],
                        hbm.at[c, pl.ds(_NS * _RT, _RREM)])


def _sc_body(src_hbm, dst_hbm, at1_hbm, at2_hbm,
             h0_hbm, h1_hbm, h2_hbm, h3_hbm, m_hbm,
             outp_hbm, denp_hbm,
             sidx, didx, sidx1, didx1, b1, b2, wres, hbuf, hbuf1,
             m_v, out_sp, gsem0, gsem1, ssem0, ssem1):
    c = lax.axis_index("c")
    s = lax.axis_index("s")
    wid = c * _NS + s
    ebase = wid * _EW
    row0 = s * _RT

    _zero_vmem(hbuf, _HQ)
    wres[pl.ds(_EW * _H, 16)] = jnp.zeros((16,), jnp.float32)
    _zero_rows(hbuf, out_sp, row0, s)
    pltpu.sync_copy(m_hbm, m_v)
    plsc.subcore_barrier()

    mv = m_v[...]
    lane = lax.iota(jnp.int32, 16)
    lo_mask = lane < 8

    @pl.loop(0, _EW, step=_B)
    def _(k):
        eb = ebase + k
        pltpu.sync_copy(src_hbm.at[pl.ds(eb, _B)], sidx)
        pltpu.sync_copy(dst_hbm.at[pl.ds(eb, _B)], didx)
        pltpu.sync_copy(at1_hbm.at[sidx], b1)
        pltpu.sync_copy(at2_hbm.at[didx], b2)

        @pl.loop(0, _B, step=2)
        def _(j):
            e1 = b1[j, :] + b2[j, :]
            e1 = jnp.maximum(e1, 0.2 * e1)
            w1 = jnp.exp(e1 - mv)
            e2 = b1[j + 1, :] + b2[j + 1, :]
            e2 = jnp.maximum(e2, 0.2 * e2)
            w2 = jnp.exp(e2 - mv)
            wres[pl.ds((k + j) * _H, 16)] = jnp.where(lo_mask, w1, w2)

    @pl.loop(0, _EW, step=_B)
    def _(k):
        eb = ebase + k
        pltpu.sync_copy(dst_hbm.at[pl.ds(eb, _B)], didx)

        @pl.loop(0, _B)
        def _(j):
            hbuf[j, pl.ds(0, 16)] = wres[pl.ds((k + j) * _H, 16)]

        pltpu.sync_copy(hbuf, out_sp.at[didx], add=True)

    plsc.subcore_barrier()
    _dump_rows(out_sp, denp_hbm, c, row0, s)
    _zero_vmem(hbuf, _HQ)
    _zero_rows(hbuf, out_sp, row0, s)
    plsc.subcore_barrier()

    si = (sidx, sidx1)
    di = (didx, didx1)
    hb = (hbuf, hbuf1)
    gs = (gsem0, gsem1)
    ss = (ssem0, ssem1)

    def _p2_compute(qt, kloc, buf):
        @pl.loop(0, _B, step=2)
        def _(e):
            wrow = wres[pl.ds((kloc + e) * _H, 16)]
            for ee in range(2):
                for hh in range(_HPQ):
                    wv = wrow[ee * _H + qt * _HPQ + hh]
                    buf[e + ee, pl.ds(hh * _D, _D)] = (
                        buf[e + ee, pl.ds(hh * _D, _D)] * wv)

    for qt, h_hbm in enumerate((h0_hbm, h1_hbm, h2_hbm, h3_hbm)):
        pltpu.sync_copy(src_hbm.at[pl.ds(ebase, _B)], si[0])
        pltpu.sync_copy(dst_hbm.at[pl.ds(ebase, _B)], di[0])
        pltpu.async_copy(h_hbm.at[si[0]], hb[0], gs[0])

        @pl.loop(0, _EW, step=2 * _B)
        def _(kk):
            for p in range(2):
                kloc = kk + p * _B
                nxt = 1 - p

                @pl.when(kloc + _B < _EW)
                def _():
                    @pl.when(kloc + (p - 1) * _B > 0)
                    def _():
                        pltpu.make_async_copy(
                            hb[nxt], out_sp.at[di[nxt]], ss[nxt]).wait()
                    pltpu.sync_copy(
                        src_hbm.at[pl.ds(ebase + kloc + _B, _B)], si[nxt])
                    pltpu.sync_copy(
                        dst_hbm.at[pl.ds(ebase + kloc + _B, _B)], di[nxt])
                    pltpu.async_copy(h_hbm.at[si[nxt]], hb[nxt], gs[nxt])

                pltpu.make_async_copy(h_hbm.at[si[p]], hb[p], gs[p]).wait()
                _p2_compute(qt, kloc, hb[p])
                pltpu.async_copy(hb[p], out_sp.at[di[p]], ss[p], add=True)

        pltpu.make_async_copy(hb[0], out_sp.at[di[0]], ss[0]).wait()
        pltpu.make_async_copy(hb[1], out_sp.at[di[1]], ss[1]).wait()
        plsc.subcore_barrier()
        _dump_rows(out_sp, outp_hbm.at[qt], c, row0, s)

        if qt < 3:
            _zero_vmem(hbuf, _HQ)
            _zero_rows(hbuf, out_sp, row0, s)
            plsc.subcore_barrier()


def _sc_edge(src, dst, at1, at2, hq, m16):
    mesh = plsc.VectorSubcoreMesh(core_axis_name="c", subcore_axis_name="s")

    k = pl.kernel(
        _sc_body,
        mesh=mesh,
        compiler_params=pltpu.CompilerParams(use_tc_tiling_on_sc=False),
        out_type=[
            jax.ShapeDtypeStruct((4, _NC, _N, _HQ), jnp.float32),
            jax.ShapeDtypeStruct((_NC, _N, _HQ), jnp.float32),
        ],
        scratch_types=[
            pltpu.VMEM((_B,), jnp.int32),
            pltpu.VMEM((_B,), jnp.int32),
            pltpu.VMEM((_B,), jnp.int32),
            pltpu.VMEM((_B,), jnp.int32),
            pltpu.VMEM((_B, _D), jnp.float32),
            pltpu.VMEM((_B, _D), jnp.float32),
            pltpu.VMEM((_EW * _H + 16,), jnp.float32),
            pltpu.VMEM((_B, _HQ), jnp.float32),
            pltpu.VMEM((_B, _HQ), jnp.float32),
            pltpu.VMEM((16,), jnp.float32),
            pltpu.VMEM_SHARED((_N, _HQ), jnp.float32),
            pltpu.SemaphoreType.DMA,
            pltpu.SemaphoreType.DMA,
            pltpu.SemaphoreType.DMA,
            pltpu.SemaphoreType.DMA,
        ],
    )
    return k(src, dst, at1, at2, *hq, m16)


def _fin_body(o00, o01, o10, o11, o20, o21, o30, o31, d0_ref, d1_ref,
              skip_ref, bias_ref, gamma_ref, beta_ref, y_ref):
    den = d0_ref[...] + d1_ref[...]
    r_row = jax.lax.broadcasted_iota(jnp.int32, (32, _HD), 0)
    r_col = jax.lax.broadcasted_iota(jnp.int32, (32, _HD), 1)
    R = (r_col // _D == r_row).astype(jnp.float32)
    drep = jnp.dot(den, R, preferred_element_type=jnp.float32)
    gat = jnp.concatenate(
        [o00[...] + o01[...], o10[...] + o11[...],
         o20[...] + o21[...], o30[...] + o31[...]], axis=1)
    pre = gat / (drep + 1e-16) + bias_ref[...] + 0.1 * skip_ref[...]
    mu = jnp.mean(pre, axis=-1, keepdims=True)
    var = jnp.mean((pre - mu) ** 2, axis=-1, keepdims=True)
    y = (pre - mu) * jax.lax.rsqrt(var + 1e-5) * gamma_ref[...] + beta_ref[...]
    y_ref[...] = jnp.where(y > 0, y, jnp.exp(jnp.minimum(y, 0.0)) - 1.0)


def _finalize(oparts, d0, d1, skip, bias, gamma, beta):
    n = skip.shape[0]
    vec = lambda i: (0, 0)
    quarter = pl.BlockSpec((_BN, _HQ), lambda i: (i, 0))
    return pl.pallas_call(
        _fin_body,
        grid=(n // _BN,),
        in_specs=[quarter] * 8 + [
            pl.BlockSpec((_BN, _HQ), lambda i: (i, 0)),
            pl.BlockSpec((_BN, _HQ), lambda i: (i, 0)),
            pl.BlockSpec((_BN, _HD), lambda i: (i, 0)),
            pl.BlockSpec((1, _HD), vec),
            pl.BlockSpec((1, _HD), vec),
            pl.BlockSpec((1, _HD), vec),
        ],
        out_specs=pl.BlockSpec((_BN, _HD), lambda i: (i, 0)),
        out_shape=jax.ShapeDtypeStruct((n, _HD), jnp.float32),
    )(*oparts, d0, d1, skip, bias, gamma, beta)


def kernel(x, edge_index, W, att_src, att_dst, bias, skip_W, gamma, beta):
    src = edge_index[0]
    dst = edge_index[1]

    Wr = W.reshape(_H, _D, _IN)
    A_src = jnp.einsum("hdk,hd->kh", Wr, att_src)
    A_dst = jnp.einsum("hdk,hd->kh", Wr, att_dst)
    A1 = jnp.concatenate([A_src, A_src], axis=1)
    A2 = jnp.concatenate([A_dst, A_dst], axis=1)
    WT = W.T
    AWq = [WT[:, q * _HQ:(q + 1) * _HQ] for q in range(4)]

    h0, h1, h2, h3, skip, at1, at2 = _dense_proj(x, AWq, skip_W.T, A1, A2)

    M = jnp.max(at1, axis=0) + jnp.max(at2, axis=0)
    M16 = jnp.maximum(M, 0.2 * M)

    outp, denp = _sc_edge(src, dst, at1, at2, (h0, h1, h2, h3), M16)

    oparts = [outp[qt, cc] for qt in range(4) for cc in range(2)]
    y = _finalize(oparts, denp[0], denp[1], skip,
                  bias.reshape(1, _HD), gamma.reshape(1, _HD),
                  beta.reshape(1, _HD))
    return y


# speedup vs baseline: 45.4954x; 1.0189x over previous
import functools
import jax
import jax.numpy as jnp
from jax import lax
from jax.experimental import pallas as pl
from jax.experimental.pallas import tpu as pltpu
from jax.experimental.pallas import tpu_sc as plsc

_N = 10000
_E = 320000
_IN = 128
_H = 8
_D = 16
_HD = _H * _D
_HQ = _HD // 8
_HPQ = _H // 8

_BN = 400
_NC = 2
_NS = 16
_NW = _NC * _NS
_EW = _E // _NW
_B = 200
_NB = _EW // _B
_RT = 624
_RZ = 104
_RREM = _N - _NS * _RT


def _proj_body(*refs):
    (x_ref, aw0, aw1, aw2, aw3, aw4, aw5, aw6, aw7, as_ref, a1_ref, a2_ref,
     h0, h1, h2, h3, h4, h5, h6, h7, sk_ref, at1_ref, at2_ref) = refs
    xb = x_ref[...]
    for aw_ref, h_ref in ((aw0, h0), (aw1, h1), (aw2, h2), (aw3, h3),
                          (aw4, h4), (aw5, h5), (aw6, h6), (aw7, h7)):
        h_ref[...] = jnp.dot(xb, aw_ref[...],
                             preferred_element_type=jnp.float32)
    sk_ref[...] = jnp.dot(xb, as_ref[...], preferred_element_type=jnp.float32)
    at1_ref[...] = jnp.dot(xb, a1_ref[...], preferred_element_type=jnp.float32)
    at2_ref[...] = jnp.dot(xb, a2_ref[...], preferred_element_type=jnp.float32)


def _dense_proj(x, AWq, AS, A1, A2):
    n = x.shape[0]
    return pl.pallas_call(
        _proj_body,
        grid=(n // _BN,),
        in_specs=[
            pl.BlockSpec((_BN, _IN), lambda i: (i, 0)),
        ] + [pl.BlockSpec((_IN, _HQ), lambda i: (0, 0))] * 8 + [
            pl.BlockSpec((_IN, _HD), lambda i: (0, 0)),
            pl.BlockSpec((_IN, 16), lambda i: (0, 0)),
            pl.BlockSpec((_IN, 16), lambda i: (0, 0)),
        ],
        out_specs=[
            pl.BlockSpec((_BN, _HQ), lambda i: (i, 0)) for _ in range(8)
        ] + [
            pl.BlockSpec((_BN, _HD), lambda i: (i, 0)),
            pl.BlockSpec((_BN, 16), lambda i: (i, 0)),
            pl.BlockSpec((_BN, 16), lambda i: (i, 0)),
        ],
        out_shape=[
            jax.ShapeDtypeStruct((n, _HQ), jnp.float32) for _ in range(8)
        ] + [
            jax.ShapeDtypeStruct((n, _HD), jnp.float32),
            jax.ShapeDtypeStruct((n, 16), jnp.float32),
            jax.ShapeDtypeStruct((n, 16), jnp.float32),
        ],
    )(x, *AWq, AS, A1, A2)


def _zero_vmem(buf, width):
    @pl.loop(0, _B)
    def _(i):
        for j in range(width // _D):
            buf[i, pl.ds(j * _D, _D)] = jnp.zeros((_D,), jnp.float32)


def _zero_rows(buf, acc, row0, s):
    for q in range(6):
        pltpu.sync_copy(buf.at[pl.ds(0, _RZ)],
                        acc.at[pl.ds(row0 + q * _RZ, _RZ)])

    @pl.when(s == _NS - 1)
    def _():
        pltpu.sync_copy(buf.at[pl.ds(0, _RREM)],
                        acc.at[pl.ds(_NS * _RT, _RREM)])


def _dump_rows(acc, hbm, c, row0, s):
    pltpu.sync_copy(acc.at[pl.ds(row0, _RT)], hbm.at[c, pl.ds(row0, _RT)])

    @pl.when(s == _NS - 1)
    def _():
        pltpu.sync_copy(acc.at[pl.ds(_NS * _RT, _RREM)],
                        hbm.at[c, pl.ds(_NS * _RT, _RREM)])


def _sc_body(src_hbm, dst_hbm, at1_hbm, at2_hbm,
             h0_hbm, h1_hbm, h2_hbm, h3_hbm, h4_hbm, h5_hbm, h6_hbm, h7_hbm,
             m_hbm,
             outp_hbm, denp_hbm,
             sidx_all, didx_all, b1, b2, wres, hbuf, hbuf1,
             m_v, out_sp, gsem0, gsem1, ssem0, ssem1):
    c = lax.axis_index("c")
    s = lax.axis_index("s")
    wid = c * _NS + s
    row0 = s * _RT

    _zero_vmem(hbuf, _HQ)
    wres[pl.ds(_EW * _H, 16)] = jnp.zeros((16,), jnp.float32)
    _zero_rows(hbuf, out_sp, row0, s)
    pltpu.sync_copy(m_hbm, m_v)

    pltpu.sync_copy(src_hbm.at[pl.ds(wid * _NB, _NB)], sidx_all)
    pltpu.sync_copy(dst_hbm.at[pl.ds(wid * _NB, _NB)], didx_all)
    plsc.subcore_barrier()

    mv = m_v[...]
    lane = lax.iota(jnp.int32, 16)
    lo_mask = lane < 8

    @pl.loop(0, _NB)
    def _(kb):
        k = kb * _B
        pltpu.sync_copy(at1_hbm.at[sidx_all.at[kb]], b1)
        pltpu.sync_copy(at2_hbm.at[didx_all.at[kb]], b2)

        @pl.loop(0, _B, step=2)
        def _(j):
            e1 = b1[j, :] + b2[j, :]
            e1 = jnp.maximum(e1, 0.2 * e1)
            w1 = jnp.exp(e1 - mv)
            e2 = b1[j + 1, :] + b2[j + 1, :]
            e2 = jnp.maximum(e2, 0.2 * e2)
            w2 = jnp.exp(e2 - mv)
            wres[pl.ds((k + j) * _H, 16)] = jnp.where(lo_mask, w1, w2)

    @pl.loop(0, _NB)
    def _(kb):
        k = kb * _B

        @pl.loop(0, _B)
        def _(j):
            hbuf[j, pl.ds(0, 16)] = wres[pl.ds((k + j) * _H, 16)]

        pltpu.sync_copy(hbuf, out_sp.at[didx_all.at[kb]], add=True)

    plsc.subcore_barrier()
    _dump_rows(out_sp, denp_hbm, c, row0, s)
    _zero_vmem(hbuf, _HQ)
    _zero_rows(hbuf, out_sp, row0, s)
    plsc.subcore_barrier()

    hb = (hbuf, hbuf1)
    gs = (gsem0, gsem1)
    ss = (ssem0, ssem1)

    def _p2_compute(qt, kloc, buf):
        @pl.loop(0, _B, step=2)
        def _(e):
            wrow = wres[pl.ds((kloc + e) * _H, 16)]
            for ee in range(2):
                for hh in range(_HPQ):
                    wv = wrow[ee * _H + qt * _HPQ + hh]
                    buf[e + ee, pl.ds(hh * _D, _D)] = (
                        buf[e + ee, pl.ds(hh * _D, _D)] * wv)

    for qt, h_hbm in enumerate((h0_hbm, h1_hbm, h2_hbm, h3_hbm,
                                h4_hbm, h5_hbm, h6_hbm, h7_hbm)):
        pltpu.async_copy(h_hbm.at[sidx_all.at[0]], hb[0], gs[0])

        @pl.loop(0, _NB, step=2)
        def _(kk):
            for p in range(2):
                kb = kk + p
                nxt = 1 - p

                @pl.when(kb + 1 < _NB)
                def _():
                    @pl.when(kb + p > 1)
                    def _():
                        pltpu.make_async_copy(
                            hb[nxt], out_sp.at[didx_all.at[kb]],
                            ss[nxt]).wait()
                    pltpu.async_copy(h_hbm.at[sidx_all.at[kb + 1]],
                                     hb[nxt], gs[nxt])

                pltpu.make_async_copy(h_hbm.at[sidx_all.at[kb]],
                                      hb[p], gs[p]).wait()
                _p2_compute(qt, kb * _B, hb[p])
                pltpu.async_copy(hb[p], out_sp.at[didx_all.at[kb]],
                                 ss[p], add=True)

        pltpu.make_async_copy(hb[0], out_sp.at[didx_all.at[0]], ss[0]).wait()
        pltpu.make_async_copy(hb[1], out_sp.at[didx_all.at[0]], ss[1]).wait()
        plsc.subcore_barrier()
        _dump_rows(out_sp, outp_hbm.at[qt], c, row0, s)

        if qt < 7:
            _zero_vmem(hbuf, _HQ)
            _zero_rows(hbuf, out_sp, row0, s)
            plsc.subcore_barrier()


def _sc_edge(src, dst, at1, at2, hq, m16):
    mesh = plsc.VectorSubcoreMesh(core_axis_name="c", subcore_axis_name="s")

    k = pl.kernel(
        _sc_body,
        mesh=mesh,
        compiler_params=pltpu.CompilerParams(use_tc_tiling_on_sc=False),
        out_type=[
            jax.ShapeDtypeStruct((8, _NC, _N, _HQ), jnp.float32),
            jax.ShapeDtypeStruct((_NC, _N, _HQ), jnp.float32),
        ],
        scratch_types=[
            pltpu.VMEM((_NB, _B), jnp.int32),
            pltpu.VMEM((_NB, _B), jnp.int32),
            pltpu.VMEM((_B, _D), jnp.float32),
            pltpu.VMEM((_B, _D), jnp.float32),
            pltpu.VMEM((_EW * _H + 16,), jnp.float32),
            pltpu.VMEM((_B, _HQ), jnp.float32),
            pltpu.VMEM((_B, _HQ), jnp.float32),
            pltpu.VMEM((16,), jnp.float32),
            pltpu.VMEM_SHARED((_N, _HQ), jnp.float32),
            pltpu.SemaphoreType.DMA,
            pltpu.SemaphoreType.DMA,
            pltpu.SemaphoreType.DMA,
            pltpu.SemaphoreType.DMA,
        ],
    )
    return k(src, dst, at1, at2, *hq, m16)


def _fin_body(*refs):
    oparts = refs[:16]
    (d0_ref, d1_ref, skip_ref, bias_ref, gamma_ref, beta_ref, y_ref) = refs[16:]
    den = d0_ref[...] + d1_ref[...]
    r_row = jax.lax.broadcasted_iota(jnp.int32, (16, _HD), 0)
    r_col = jax.lax.broadcasted_iota(jnp.int32, (16, _HD), 1)
    R = (r_col // _D == r_row).astype(jnp.float32)
    drep = jnp.dot(den, R, preferred_element_type=jnp.float32)
    gat = jnp.concatenate(
        [oparts[2 * q][...] + oparts[2 * q + 1][...] for q in range(8)],
        axis=1)
    pre = gat / (drep + 1e-16) + bias_ref[...] + 0.1 * skip_ref[...]
    mu = jnp.mean(pre, axis=-1, keepdims=True)
    var = jnp.mean((pre - mu) ** 2, axis=-1, keepdims=True)
    y = (pre - mu) * jax.lax.rsqrt(var + 1e-5) * gamma_ref[...] + beta_ref[...]
    y_ref[...] = jnp.where(y > 0, y, jnp.exp(jnp.minimum(y, 0.0)) - 1.0)


def _finalize(oparts, d0, d1, skip, bias, gamma, beta):
    n = skip.shape[0]
    vec = lambda i: (0, 0)
    quarter = pl.BlockSpec((_BN, _HQ), lambda i: (i, 0))
    return pl.pallas_call(
        _fin_body,
        grid=(n // _BN,),
        in_specs=[quarter] * 16 + [
            pl.BlockSpec((_BN, _HQ), lambda i: (i, 0)),
            pl.BlockSpec((_BN, _HQ), lambda i: (i, 0)),
            pl.BlockSpec((_BN, _HD), lambda i: (i, 0)),
            pl.BlockSpec((1, _HD), vec),
            pl.BlockSpec((1, _HD), vec),
            pl.BlockSpec((1, _HD), vec),
        ],
        out_specs=pl.BlockSpec((_BN, _HD), lambda i: (i, 0)),
        out_shape=jax.ShapeDtypeStruct((n, _HD), jnp.float32),
    )(*oparts, d0, d1, skip, bias, gamma, beta)


def kernel(x, edge_index, W, att_src, att_dst, bias, skip_W, gamma, beta):
    src = edge_index[0]
    dst = edge_index[1]

    Wr = W.reshape(_H, _D, _IN)
    A_src = jnp.einsum("hdk,hd->kh", Wr, att_src)
    A_dst = jnp.einsum("hdk,hd->kh", Wr, att_dst)
    A1 = jnp.concatenate([A_src, A_src], axis=1)
    A2 = jnp.concatenate([A_dst, A_dst], axis=1)
    WT = W.T
    AWq = [WT[:, q * _HQ:(q + 1) * _HQ] for q in range(8)]

    (h0, h1, h2, h3, h4, h5, h6, h7,
     skip, at1, at2) = _dense_proj(x, AWq, skip_W.T, A1, A2)

    M = jnp.max(at1, axis=0) + jnp.max(at2, axis=0)
    M16 = jnp.maximum(M, 0.2 * M)

    src2 = src.reshape(_NW * _NB, _B)
    dst2 = dst.reshape(_NW * _NB, _B)
    outp, denp = _sc_edge(src2, dst2, at1, at2,
                          (h0, h1, h2, h3, h4, h5, h6, h7), M16)

    oparts = [outp[qt, cc] for qt in range(8) for cc in range(2)]
    y = _finalize(oparts, denp[0], denp[1], skip,
                  bias.reshape(1, _HD), gamma.reshape(1, _HD),
                  beta.reshape(1, _HD))
    return y


# speedup vs baseline: 54.6953x; 1.2022x over previous
import functools
import jax
import jax.numpy as jnp
from jax import lax
from jax.experimental import pallas as pl
from jax.experimental.pallas import tpu as pltpu
from jax.experimental.pallas import tpu_sc as plsc

_N = 10000
_E = 320000
_IN = 128
_H = 8
_D = 16
_HD = _H * _D
_HQ = _HD // 8
_HPQ = _H // 8

_BN = 400
_NC = 2
_NS = 16
_NW = _NC * _NS
_EW = _E // _NW
_B = 200
_NB = _EW // _B
_RT = 624
_RZ = 104
_RREM = _N - _NS * _RT


def _proj_body(*refs):
    (x_ref, aw0, aw1, aw2, aw3, aw4, aw5, aw6, aw7, as_ref, a1_ref, a2_ref,
     h0, h1, h2, h3, h4, h5, h6, h7, sk_ref, at1_ref, at2_ref) = refs
    xb = x_ref[...]
    for aw_ref, h_ref in ((aw0, h0), (aw1, h1), (aw2, h2), (aw3, h3),
                          (aw4, h4), (aw5, h5), (aw6, h6), (aw7, h7)):
        h_ref[...] = jnp.dot(xb, aw_ref[...],
                             preferred_element_type=jnp.float32)
    sk_ref[...] = jnp.dot(xb, as_ref[...], preferred_element_type=jnp.float32)
    at1_ref[...] = jnp.dot(xb, a1_ref[...], preferred_element_type=jnp.float32)
    at2_ref[...] = jnp.dot(xb, a2_ref[...], preferred_element_type=jnp.float32)


def _dense_proj(x, AWq, AS, A1, A2):
    n = x.shape[0]
    return pl.pallas_call(
        _proj_body,
        grid=(n // _BN,),
        in_specs=[
            pl.BlockSpec((_BN, _IN), lambda i: (i, 0)),
        ] + [pl.BlockSpec((_IN, _HQ), lambda i: (0, 0))] * 8 + [
            pl.BlockSpec((_IN, _HD), lambda i: (0, 0)),
            pl.BlockSpec((_IN, 16), lambda i: (0, 0)),
            pl.BlockSpec((_IN, 16), lambda i: (0, 0)),
        ],
        out_specs=[
            pl.BlockSpec((_BN, _HQ), lambda i: (i, 0)) for _ in range(8)
        ] + [
            pl.BlockSpec((_BN, _HD), lambda i: (i, 0)),
            pl.BlockSpec((_BN, 16), lambda i: (i, 0)),
            pl.BlockSpec((_BN, 16), lambda i: (i, 0)),
        ],
        out_shape=[
            jax.ShapeDtypeStruct((n, _HQ), jnp.float32) for _ in range(8)
        ] + [
            jax.ShapeDtypeStruct((n, _HD), jnp.float32),
            jax.ShapeDtypeStruct((n, 16), jnp.float32),
            jax.ShapeDtypeStruct((n, 16), jnp.float32),
        ],
    )(x, *AWq, AS, A1, A2)


def _zero_vmem(buf, width):
    @pl.loop(0, _B)
    def _(i):
        for j in range(width // _D):
            buf[i, pl.ds(j * _D, _D)] = jnp.zeros((_D,), jnp.float32)


def _zero_rows(buf, acc, row0, s):
    for q in range(6):
        pltpu.sync_copy(buf.at[pl.ds(0, _RZ)],
                        acc.at[pl.ds(row0 + q * _RZ, _RZ)])

    @pl.when(s == _NS - 1)
    def _():
        pltpu.sync_copy(buf.at[pl.ds(0, _RREM)],
                        acc.at[pl.ds(_NS * _RT, _RREM)])


def _dump_rows(acc, hbm, c, row0, s):
    pltpu.sync_copy(acc.at[pl.ds(row0, _RT)], hbm.at[c, pl.ds(row0, _RT)])

    @pl.when(s == _NS - 1)
    def _():
        pltpu.sync_copy(acc.at[pl.ds(_NS * _RT, _RREM)],
                        hbm.at[c, pl.ds(_NS * _RT, _RREM)])


def _sc_body(src_hbm, dst_hbm, at1_hbm, at2_hbm,
             h0_hbm, h1_hbm, h2_hbm, h3_hbm, h4_hbm, h5_hbm, h6_hbm, h7_hbm,
             m_hbm,
             outp_hbm, denp_hbm,
             sidx_all, didx_all, b1, b2, b1x, b2x, wres, hbuf, hbuf1,
             m_v, out_sp, gsem0, gsem1, ssem0, ssem1):
    c = lax.axis_index("c")
    s = lax.axis_index("s")
    wid = c * _NS + s
    row0 = s * _RT

    _zero_vmem(hbuf, _HQ)
    wres[pl.ds(_EW * _H, 16)] = jnp.zeros((16,), jnp.float32)
    _zero_rows(hbuf, out_sp, row0, s)
    pltpu.sync_copy(m_hbm, m_v)

    pltpu.sync_copy(src_hbm.at[pl.ds(wid * _NB, _NB)], sidx_all)
    pltpu.sync_copy(dst_hbm.at[pl.ds(wid * _NB, _NB)], didx_all)
    plsc.subcore_barrier()

    mv = m_v[...]
    lane = lax.iota(jnp.int32, 16)
    lo_mask = lane < 8

    bb1 = (b1, b1x)
    bb2 = (b2, b2x)

    hb = (hbuf, hbuf1)
    gs = (gsem0, gsem1)
    ss = (ssem0, ssem1)

    pltpu.async_copy(at1_hbm.at[sidx_all.at[0]], bb1[0], gs[0])
    pltpu.async_copy(at2_hbm.at[didx_all.at[0]], bb2[0], gs[0])

    @pl.loop(0, _NB, step=2)
    def _(kk):
      for p in range(2):
        kb = kk + p
        nxt = 1 - p
        k = kb * _B

        @pl.when(kb + 1 < _NB)
        def _():
            pltpu.async_copy(at1_hbm.at[sidx_all.at[kb + 1]], bb1[nxt],
                             gs[nxt])
            pltpu.async_copy(at2_hbm.at[didx_all.at[kb + 1]], bb2[nxt],
                             gs[nxt])

        pltpu.make_async_copy(at1_hbm.at[sidx_all.at[kb]], bb1[p],
                              gs[p]).wait()
        pltpu.make_async_copy(at2_hbm.at[didx_all.at[kb]], bb2[p],
                              gs[p]).wait()

        @pl.when(kb > 1)
        def _():
            pltpu.make_async_copy(hb[p], out_sp.at[didx_all.at[kb]],
                                  ss[p]).wait()

        @pl.loop(0, _B, step=4)
        def _(j):
            for u in range(2):
                jj = j + 2 * u
                e1 = bb1[p][jj, :] + bb2[p][jj, :]
                e1 = jnp.maximum(e1, 0.2 * e1)
                w1 = jnp.exp(e1 - mv)
                e2 = bb1[p][jj + 1, :] + bb2[p][jj + 1, :]
                e2 = jnp.maximum(e2, 0.2 * e2)
                w2 = jnp.exp(e2 - mv)
                hb[p][jj, :] = w1
                hb[p][jj + 1, :] = w2
                wres[pl.ds((k + jj) * _H, 16)] = jnp.where(lo_mask, w1, w2)

        pltpu.async_copy(hb[p], out_sp.at[didx_all.at[kb]], ss[p], add=True)

    pltpu.make_async_copy(hb[0], out_sp.at[didx_all.at[0]], ss[0]).wait()
    pltpu.make_async_copy(hb[1], out_sp.at[didx_all.at[0]], ss[1]).wait()

    plsc.subcore_barrier()
    _dump_rows(out_sp, denp_hbm, c, row0, s)
    _zero_vmem(hbuf, _HQ)
    _zero_rows(hbuf, out_sp, row0, s)
    plsc.subcore_barrier()

    def _p2_compute(qt, kloc, buf):
        @pl.loop(0, _B, step=8)
        def _(e):
            for u in range(4):
                e0 = e + 2 * u
                wrow = wres[pl.ds((kloc + e0) * _H, 16)]
                for ee in range(2):
                    wv = wrow[ee * _H + qt]
                    buf[e0 + ee, :] = buf[e0 + ee, :] * wv

    for qt, h_hbm in enumerate((h0_hbm, h1_hbm, h2_hbm, h3_hbm,
                                h4_hbm, h5_hbm, h6_hbm, h7_hbm)):
        pltpu.async_copy(h_hbm.at[sidx_all.at[0]], hb[0], gs[0])

        @pl.loop(0, _NB, step=2)
        def _(kk):
            for p in range(2):
                kb = kk + p
                nxt = 1 - p

                @pl.when(kb + 1 < _NB)
                def _():
                    @pl.when(kb + p > 1)
                    def _():
                        pltpu.make_async_copy(
                            hb[nxt], out_sp.at[didx_all.at[kb]],
                            ss[nxt]).wait()
                    pltpu.async_copy(h_hbm.at[sidx_all.at[kb + 1]],
                                     hb[nxt], gs[nxt])

                pltpu.make_async_copy(h_hbm.at[sidx_all.at[kb]],
                                      hb[p], gs[p]).wait()
                _p2_compute(qt, kb * _B, hb[p])
                pltpu.async_copy(hb[p], out_sp.at[didx_all.at[kb]],
                                 ss[p], add=True)

        pltpu.make_async_copy(hb[0], out_sp.at[didx_all.at[0]], ss[0]).wait()
        pltpu.make_async_copy(hb[1], out_sp.at[didx_all.at[0]], ss[1]).wait()
        plsc.subcore_barrier()
        _dump_rows(out_sp, outp_hbm.at[qt], c, row0, s)

        if qt < 7:
            _zero_vmem(hbuf, _HQ)
            _zero_rows(hbuf, out_sp, row0, s)
            plsc.subcore_barrier()


def _sc_edge(src, dst, at1, at2, hq, m16):
    mesh = plsc.VectorSubcoreMesh(core_axis_name="c", subcore_axis_name="s")

    k = pl.kernel(
        _sc_body,
        mesh=mesh,
        compiler_params=pltpu.CompilerParams(use_tc_tiling_on_sc=False),
        out_type=[
            jax.ShapeDtypeStruct((8, _NC, _N, _HQ), jnp.float32),
            jax.ShapeDtypeStruct((_NC, _N, _HQ), jnp.float32),
        ],
        scratch_types=[
            pltpu.VMEM((_NB, _B), jnp.int32),
            pltpu.VMEM((_NB, _B), jnp.int32),
            pltpu.VMEM((_B, _D), jnp.float32),
            pltpu.VMEM((_B, _D), jnp.float32),
            pltpu.VMEM((_B, _D), jnp.float32),
            pltpu.VMEM((_B, _D), jnp.float32),
            pltpu.VMEM((_EW * _H + 16,), jnp.float32),
            pltpu.VMEM((_B, _HQ), jnp.float32),
            pltpu.VMEM((_B, _HQ), jnp.float32),
            pltpu.VMEM((16,), jnp.float32),
            pltpu.VMEM_SHARED((_N, _HQ), jnp.float32),
            pltpu.SemaphoreType.DMA,
            pltpu.SemaphoreType.DMA,
            pltpu.SemaphoreType.DMA,
            pltpu.SemaphoreType.DMA,
        ],
    )
    return k(src, dst, at1, at2, *hq, m16)


def _fin_body(*refs):
    oparts = refs[:16]
    (d0_ref, d1_ref, skip_ref, bias_ref, gamma_ref, beta_ref, y_ref) = refs[16:]
    den = d0_ref[...] + d1_ref[...]
    r_row = jax.lax.broadcasted_iota(jnp.int32, (16, _HD), 0)
    r_col = jax.lax.broadcasted_iota(jnp.int32, (16, _HD), 1)
    R = (r_col // _D == r_row).astype(jnp.float32)
    drep = jnp.dot(den, R, preferred_element_type=jnp.float32)
    gat = jnp.concatenate(
        [oparts[2 * q][...] + oparts[2 * q + 1][...] for q in range(8)],
        axis=1)
    pre = gat / (drep + 1e-16) + bias_ref[...] + 0.1 * skip_ref[...]
    mu = jnp.mean(pre, axis=-1, keepdims=True)
    var = jnp.mean((pre - mu) ** 2, axis=-1, keepdims=True)
    y = (pre - mu) * jax.lax.rsqrt(var + 1e-5) * gamma_ref[...] + beta_ref[...]
    y_ref[...] = jnp.where(y > 0, y, jnp.exp(jnp.minimum(y, 0.0)) - 1.0)


def _finalize(oparts, d0, d1, skip, bias, gamma, beta):
    n = skip.shape[0]
    vec = lambda i: (0, 0)
    quarter = pl.BlockSpec((_BN, _HQ), lambda i: (i, 0))
    return pl.pallas_call(
        _fin_body,
        grid=(n // _BN,),
        in_specs=[quarter] * 16 + [
            pl.BlockSpec((_BN, _HQ), lambda i: (i, 0)),
            pl.BlockSpec((_BN, _HQ), lambda i: (i, 0)),
            pl.BlockSpec((_BN, _HD), lambda i: (i, 0)),
            pl.BlockSpec((1, _HD), vec),
            pl.BlockSpec((1, _HD), vec),
            pl.BlockSpec((1, _HD), vec),
        ],
        out_specs=pl.BlockSpec((_BN, _HD), lambda i: (i, 0)),
        out_shape=jax.ShapeDtypeStruct((n, _HD), jnp.float32),
    )(*oparts, d0, d1, skip, bias, gamma, beta)


def kernel(x, edge_index, W, att_src, att_dst, bias, skip_W, gamma, beta):
    src = edge_index[0]
    dst = edge_index[1]

    Wr = W.reshape(_H, _D, _IN)
    A_src = jnp.einsum("hdk,hd->kh", Wr, att_src)
    A_dst = jnp.einsum("hdk,hd->kh", Wr, att_dst)
    A1 = jnp.concatenate([A_src, A_src], axis=1)
    A2 = jnp.concatenate([A_dst, A_dst], axis=1)
    WT = W.T
    AWq = [WT[:, q * _HQ:(q + 1) * _HQ] for q in range(8)]

    (h0, h1, h2, h3, h4, h5, h6, h7,
     skip, at1, at2) = _dense_proj(x, AWq, skip_W.T, A1, A2)

    M = jnp.max(at1, axis=0) + jnp.max(at2, axis=0)
    M16 = jnp.maximum(M, 0.2 * M)

    src2 = src.reshape(_NW * _NB, _B)
    dst2 = dst.reshape(_NW * _NB, _B)
    outp, denp = _sc_edge(src2, dst2, at1, at2,
                          (h0, h1, h2, h3, h4, h5, h6, h7), M16)

    oparts = [outp[qt, cc] for qt in range(8) for cc in range(2)]
    y = _finalize(oparts, denp[0], denp[1], skip,
                  bias.reshape(1, _HD), gamma.reshape(1, _HD),
                  beta.reshape(1, _HD))
    return y
